# baseline (speedup 1.0000x reference)
import jax
import jax.numpy as jnp
from jax import lax
from jax.experimental import pallas as pl
from jax.experimental.pallas import tpu as pltpu
from jax.experimental.pallas import tpu_sc as plsc

N = 10000
E = 320000
D = 128

NC = 2
NS = 16
NW = NC * NS

GCHUNK = 80
UNIT = NW * GCHUNK
UNITS = [9, 13, 19, 28, 37, 19]
UBASE = [sum(UNITS[:i]) for i in range(len(UNITS))]
CH = len(UNITS)

NP = 10240
ROWS_PER_SUB = NP // NS

BLK = UNIT

_f32 = jnp.float32


def _make_gather_body(u):
    def body(table, sidx3, didx3, vi_hbm, vj_hbm,
             idx_vs, idx_vd, rows0, rows1, sg0, sg1, sw0, sw1):
        c = lax.axis_index("c")
        s = lax.axis_index("s")
        wid = c * NS + s
        base = wid * u * GCHUNK
        pltpu.sync_copy(sidx3.at[wid], idx_vs)
        pltpu.sync_copy(didx3.at[wid], idx_vd)

        def half(idx_v, out_hbm):
            def pair(j, carry):
                k0 = 2 * j
                k1 = k0 + 1
                g0 = pltpu.async_copy(table.at[idx_v.at[k0]], rows0, sg0)
                g1 = pltpu.async_copy(table.at[idx_v.at[k1]], rows1, sg1)
                g0.wait()
                w0 = pltpu.async_copy(
                    rows0, out_hbm.at[pl.ds(base + k0 * GCHUNK, GCHUNK)],
                    sw0)
                g1.wait()
                w1 = pltpu.async_copy(
                    rows1, out_hbm.at[pl.ds(base + k1 * GCHUNK, GCHUNK)],
                    sw1)
                w0.wait()
                w1.wait()
                return carry

            lax.fori_loop(0, u // 2, pair, 0)
            if u % 2 == 1:
                kt = u - 1
                pltpu.async_copy(table.at[idx_v.at[kt]], rows0, sg0).wait()
                pltpu.sync_copy(
                    rows0, out_hbm.at[pl.ds(base + kt * GCHUNK, GCHUNK)])

        half(idx_vs, vi_hbm)
        half(idx_vd, vj_hbm)

    return body


def _sc_gather(node_feat, sidx3, didx3, u):
    ech = u * UNIT
    return pl.kernel(
        _make_gather_body(u),
        out_type=[jax.ShapeDtypeStruct((ech, D), _f32),
                  jax.ShapeDtypeStruct((ech, D), _f32)],
        mesh=plsc.VectorSubcoreMesh(core_axis_name="c", subcore_axis_name="s"),
        scratch_types=[
            pltpu.VMEM((u, GCHUNK), jnp.int32),
            pltpu.VMEM((u, GCHUNK), jnp.int32),
            pltpu.VMEM((GCHUNK, D), _f32),
            pltpu.VMEM((GCHUNK, D), _f32),
            pltpu.SemaphoreType.DMA,
            pltpu.SemaphoreType.DMA,
            pltpu.SemaphoreType.DMA,
            pltpu.SemaphoreType.DMA,
        ],
    )(node_feat, sidx3, didx3)


def _make_scatter_body(units, zero_init):
    nmess = len(units)

    def body(*refs):
        mess_refs = refs[:nmess]
        dst_refs = refs[nmess:2 * nmess]
        if zero_init:
            (out_hbm, idx_v, rows0, rows1, acc,
             sl0, sl1, ss0, ss1, zbuf) = refs[2 * nmess:]
        else:
            (init, out_hbm, idx_v, rows0, rows1, acc,
             sl0, sl1, ss0, ss1) = refs[2 * nmess:]
        c = lax.axis_index("c")
        s = lax.axis_index("s")

        if zero_init:
            zv = jnp.zeros((16,), _f32)

            def zrow(r, carry):
                for cc in range(D // 16):
                    zbuf[r, pl.ds(cc * 16, 16)] = zv
                return carry

            lax.fori_loop(0, GCHUNK, zrow, 0)

            def zcp(t, carry):
                pltpu.sync_copy(
                    zbuf,
                    acc.at[pl.ds(s * ROWS_PER_SUB + t * GCHUNK, GCHUNK)])
                return carry

            lax.fori_loop(0, ROWS_PER_SUB // GCHUNK, zcp, 0)
        else:
            @pl.when(s == 0)
            def _init():
                pltpu.sync_copy(init.at[c], acc)

        plsc.subcore_barrier()

        wid = c * NS + s
        for mi in range(nmess):
            u = units[mi]
            mref = mess_refs[mi]
            lbase = wid * u * GCHUNK
            pltpu.sync_copy(dst_refs[mi].at[wid], idx_v.at[pl.ds(0, u)])

            def pair(j, carry, mref=mref, lbase=lbase):
                k0 = 2 * j
                k1 = k0 + 1
                l0 = pltpu.async_copy(
                    mref.at[pl.ds(lbase + k0 * GCHUNK, GCHUNK)], rows0, sl0)
                l1 = pltpu.async_copy(
                    mref.at[pl.ds(lbase + k1 * GCHUNK, GCHUNK)], rows1, sl1)
                l0.wait()
                s0 = pltpu.async_copy(rows0, acc.at[idx_v.at[k0]], ss0,
                                      add=True)
                l1.wait()
                s1 = pltpu.async_copy(rows1, acc.at[idx_v.at[k1]], ss1,
                                      add=True)
                s0.wait()
                s1.wait()
                return carry

            lax.fori_loop(0, u // 2, pair, 0)
            if u % 2 == 1:
                kt = u - 1
                pltpu.sync_copy(
                    mref.at[pl.ds(lbase + kt * GCHUNK, GCHUNK)], rows0)
                pltpu.sync_copy(rows0, acc.at[idx_v.at[kt]], add=True)

        plsc.subcore_barrier()
        rbase = s * ROWS_PER_SUB
        pltpu.sync_copy(acc.at[pl.ds(rbase, ROWS_PER_SUB)],
                        out_hbm.at[c, pl.ds(rbase, ROWS_PER_SUB)])

    return body


def _sc_scatter(mess_list, dst3_list, units, init):
    zero_init = init is None
    max_u = max(units)
    scratch = [
        pltpu.VMEM((max_u, GCHUNK), jnp.int32),
        pltpu.VMEM((GCHUNK, D), _f32),
        pltpu.VMEM((GCHUNK, D), _f32),
        pltpu.VMEM_SHARED((NP, D), _f32),
        pltpu.SemaphoreType.DMA,
        pltpu.SemaphoreType.DMA,
        pltpu.SemaphoreType.DMA,
        pltpu.SemaphoreType.DMA,
    ]
    args = list(mess_list) + list(dst3_list)
    if zero_init:
        scratch.append(pltpu.VMEM((GCHUNK, D), _f32))
    else:
        args.append(init)
    return pl.kernel(
        _make_scatter_body(units, zero_init),
        out_type=jax.ShapeDtypeStruct((NC, NP, D), _f32),
        mesh=plsc.VectorSubcoreMesh(core_axis_name="c", subcore_axis_name="s"),
        scratch_types=scratch,
    )(*args)


def _edge_body(vi, vj, ef, rbf, wsrc, wdst, wee, wen, w2, bias, rbfw,
               new_e, mess):
    f32 = jnp.float32
    efb = ef[:]
    pvi = jnp.dot(vi[:], wsrc[:], preferred_element_type=f32)
    pvj = jnp.dot(vj[:], wdst[:], preferred_element_type=f32)
    basep = pvi + pvj
    pe = jnp.dot(efb, wee[:], preferred_element_type=f32)
    r = lax.dot_general(rbf[:], rbfw[:], (((0,), (0,)), ((), ())),
                        preferred_element_type=f32)

    e_h1 = jax.nn.silu(basep[:, 0:D] + pe[:, 0:D] + bias[0])
    e_g1 = jax.nn.silu(basep[:, D:2 * D] + pe[:, D:2 * D] + bias[2])
    e_h2 = jax.nn.silu(jnp.dot(e_h1, w2[0], preferred_element_type=f32)
                       + bias[1])
    e_g = jax.nn.sigmoid(jnp.dot(e_g1, w2[1], preferred_element_type=f32)
                         + bias[3])
    ne = efb + e_h2 * e_g * r[:, 0:D]
    new_e[:] = ne

    pne = jnp.dot(ne, wen[:], preferred_element_type=f32)
    n_h1 = jax.nn.silu(basep[:, 2 * D:3 * D] + pne[:, 0:D] + bias[4])
    n_g1 = jax.nn.silu(basep[:, 3 * D:4 * D] + pne[:, D:2 * D] + bias[6])
    n_h2 = jax.nn.silu(jnp.dot(n_h1, w2[2], preferred_element_type=f32)
                       + bias[5])
    n_g = jax.nn.sigmoid(jnp.dot(n_g1, w2[3], preferred_element_type=f32)
                         + bias[7])
    mess[:] = n_h2 * n_g * r[:, D:2 * D]


def _edge_body_alias(ne_in, vi, vj, ef, rbf, wsrc, wdst, wee, wen, w2,
                     bias, rbfw, new_e, mess):
    del ne_in
    _edge_body(vi, vj, ef, rbf, wsrc, wdst, wee, wen, w2, bias, rbfw,
               new_e, mess)


def _weight_specs():
    whole2 = lambda i: (0, 0)
    whole3 = lambda i: (0, 0, 0)
    return [
        pl.BlockSpec((D, 4 * D), whole2),
        pl.BlockSpec((D, 4 * D), whole2),
        pl.BlockSpec((D, 2 * D), whole2),
        pl.BlockSpec((D, 2 * D), whole2),
        pl.BlockSpec((4, D, D), whole3),
        pl.BlockSpec((8, D), whole2),
        pl.BlockSpec((9, 2 * D), whole2),
    ]


def _tc_edge_chunk(ne_buf, vi, vj, ef, rbft, weights, ch):
    u = UNITS[ch]
    ub = UBASE[ch]
    row = lambda i: (i, 0)
    rowc = lambda i, ub=ub: (ub + i, 0)
    colc = lambda i, ub=ub: (0, ub + i)
    first = ne_buf is None
    body = _edge_body if first else _edge_body_alias
    in_specs = [
        pl.BlockSpec((BLK, D), row),
        pl.BlockSpec((BLK, D), row),
        pl.BlockSpec((BLK, D), rowc),
        pl.BlockSpec((9, BLK), colc),
    ] + _weight_specs()
    args = [vi, vj, ef, rbft] + list(weights)
    aliases = {}
    if not first:
        in_specs = [pl.BlockSpec(memory_space=pltpu.MemorySpace.HBM)] \
            + in_specs
        args = [ne_buf] + args
        aliases = {0: 0}
    return pl.pallas_call(
        body,
        grid=(u,),
        in_specs=in_specs,
        out_specs=[
            pl.BlockSpec((BLK, D), rowc),
            pl.BlockSpec((BLK, D), row),
        ],
        out_shape=[
            jax.ShapeDtypeStruct((E, D), _f32),
            jax.ShapeDtypeStruct((u * UNIT, D), _f32),
        ],
        input_output_aliases=aliases,
        compiler_params=pltpu.CompilerParams(
            dimension_semantics=("arbitrary",)),
    )(*args)


def _combine_body(nf, p, out):
    out[:] = nf[:] + p[0] + p[1]


def _tc_combine(node_feat, partials):
    blk = 1000
    return pl.pallas_call(
        _combine_body,
        grid=(N // blk,),
        in_specs=[
            pl.BlockSpec((blk, D), lambda i: (i, 0)),
            pl.BlockSpec((NC, blk, D), lambda i: (0, i, 0)),
        ],
        out_specs=pl.BlockSpec((blk, D), lambda i: (i, 0)),
        out_shape=jax.ShapeDtypeStruct((N, D), _f32),
    )(node_feat, partials)


def kernel(node_feat, edge_feat, rbf, state_feat, edge_index,
           ew1, eb1, ew2, eb2, egw1, egb1, egw2, egb2, edge_rbf_w,
           nw1, nb1, nw2, nb2, ngw1, ngb1, ngw2, ngb2, node_rbf_w):
    src = edge_index[0].astype(jnp.int32)
    dst = edge_index[1].astype(jnp.int32)

    def chunk_idx(flat, ch):
        eb = UBASE[ch] * UNIT
        u = UNITS[ch]
        return lax.dynamic_slice(flat, (eb,), (u * UNIT,)).reshape(
            NW, u, GCHUNK)

    wsrc = jnp.concatenate(
        [ew1[:D], egw1[:D], nw1[:D], ngw1[:D]], axis=1)
    wdst = jnp.concatenate(
        [ew1[D:2 * D], egw1[D:2 * D], nw1[D:2 * D], ngw1[D:2 * D]], axis=1)
    wee = jnp.concatenate([ew1[2 * D:], egw1[2 * D:]], axis=1)
    wen = jnp.concatenate([nw1[2 * D:], ngw1[2 * D:]], axis=1)
    w2 = jnp.stack([ew2, egw2, nw2, ngw2])
    bias = jnp.stack([eb1, eb2, egb1, egb2, nb1, nb2, ngb1, ngb2])
    rbfw = jnp.concatenate([edge_rbf_w, node_rbf_w], axis=1)
    rbft = rbf.T
    weights = (wsrc, wdst, wee, wen, w2, bias, rbfw)

    ne_buf = None
    mess_chunks = []
    dst3s = []
    for ch in range(CH):
        vi, vj = _sc_gather(node_feat, chunk_idx(src, ch),
                            chunk_idx(dst, ch), UNITS[ch])
        ne_buf, m = _tc_edge_chunk(ne_buf, vi, vj,
                                   edge_feat, rbft, weights, ch)
        mess_chunks.append(m)
        dst3s.append(chunk_idx(dst, ch))

    pa = _sc_scatter(mess_chunks[:CH - 1], dst3s[:CH - 1],
                     UNITS[:CH - 1], None)
    pb = _sc_scatter(mess_chunks[CH - 1:], dst3s[CH - 1:],
                     UNITS[CH - 1:], pa)
    new_v = _tc_combine(node_feat, pb)
    return ne_buf, new_v, state_feat


# speedup vs baseline: 1.0676x; 1.0676x over previous
import jax
import jax.numpy as jnp
from jax import lax
from jax.experimental import pallas as pl
from jax.experimental.pallas import tpu as pltpu
from jax.experimental.pallas import tpu_sc as plsc

N = 10000
E = 320000
D = 128

NC = 2
NS = 16
NW = NC * NS

GCHUNK = 80
UNIT = NW * GCHUNK
UNITS = [9, 13, 19, 28, 37, 19]
UBASE = [sum(UNITS[:i]) for i in range(len(UNITS))]
CH = len(UNITS)

NP = 10240
ROWS_PER_SUB = NP // NS

BLK = UNIT

_f32 = jnp.float32


def _make_gather_body(u):
    def body(table, sidx3, didx3, vi_hbm, vj_hbm,
             idx_vs, idx_vd, rows0, rows1, sg0, sg1, sw0, sw1):
        c = lax.axis_index("c")
        s = lax.axis_index("s")
        wid = c * NS + s
        base = wid * u * GCHUNK
        pltpu.sync_copy(sidx3.at[wid], idx_vs)
        pltpu.sync_copy(didx3.at[wid], idx_vd)

        def half(idx_v, out_hbm):
            def pair(j, carry):
                k0 = 2 * j
                k1 = k0 + 1
                g0 = pltpu.async_copy(table.at[idx_v.at[k0]], rows0, sg0)
                g1 = pltpu.async_copy(table.at[idx_v.at[k1]], rows1, sg1)
                g0.wait()
                w0 = pltpu.async_copy(
                    rows0, out_hbm.at[pl.ds(base + k0 * GCHUNK, GCHUNK)],
                    sw0)
                g1.wait()
                w1 = pltpu.async_copy(
                    rows1, out_hbm.at[pl.ds(base + k1 * GCHUNK, GCHUNK)],
                    sw1)
                w0.wait()
                w1.wait()
                return carry

            lax.fori_loop(0, u // 2, pair, 0)
            if u % 2 == 1:
                kt = u - 1
                pltpu.async_copy(table.at[idx_v.at[kt]], rows0, sg0).wait()
                pltpu.sync_copy(
                    rows0, out_hbm.at[pl.ds(base + kt * GCHUNK, GCHUNK)])

        half(idx_vs, vi_hbm)
        half(idx_vd, vj_hbm)

    return body


def _sc_gather(node_feat, sidx3, didx3, u):
    ech = u * UNIT
    return pl.kernel(
        _make_gather_body(u),
        out_type=[jax.ShapeDtypeStruct((ech, D), _f32),
                  jax.ShapeDtypeStruct((ech, D), _f32)],
        mesh=plsc.VectorSubcoreMesh(core_axis_name="c", subcore_axis_name="s"),
        scratch_types=[
            pltpu.VMEM((u, GCHUNK), jnp.int32),
            pltpu.VMEM((u, GCHUNK), jnp.int32),
            pltpu.VMEM((GCHUNK, D), _f32),
            pltpu.VMEM((GCHUNK, D), _f32),
            pltpu.SemaphoreType.DMA,
            pltpu.SemaphoreType.DMA,
            pltpu.SemaphoreType.DMA,
            pltpu.SemaphoreType.DMA,
        ],
    )(node_feat, sidx3, didx3)


def _make_scatter_body(units, zero_init):
    nmess = len(units)

    def body(*refs):
        mess_refs = refs[:nmess]
        dst_refs = refs[nmess:2 * nmess]
        if zero_init:
            (out_hbm, idx_v, rows0, rows1, acc,
             sl0, sl1, ss0, ss1, zbuf) = refs[2 * nmess:]
        else:
            (init, out_hbm, idx_v, rows0, rows1, acc,
             sl0, sl1, ss0, ss1) = refs[2 * nmess:]
        c = lax.axis_index("c")
        s = lax.axis_index("s")

        if zero_init:
            zv = jnp.zeros((16,), _f32)

            def zrow(r, carry):
                for cc in range(D // 16):
                    zbuf[r, pl.ds(cc * 16, 16)] = zv
                return carry

            lax.fori_loop(0, GCHUNK, zrow, 0)

            def zcp(t, carry):
                pltpu.sync_copy(
                    zbuf,
                    acc.at[pl.ds(s * ROWS_PER_SUB + t * GCHUNK, GCHUNK)])
                return carry

            lax.fori_loop(0, ROWS_PER_SUB // GCHUNK, zcp, 0)
        else:
            @pl.when(s == 0)
            def _init():
                pltpu.sync_copy(init.at[c], acc)

        plsc.subcore_barrier()

        wid = c * NS + s
        for mi in range(nmess):
            u = units[mi]
            mref = mess_refs[mi]
            lbase = wid * u * GCHUNK
            pltpu.sync_copy(dst_refs[mi].at[wid], idx_v.at[pl.ds(0, u)])

            def pair(j, carry, mref=mref, lbase=lbase):
                k0 = 2 * j
                k1 = k0 + 1
                l0 = pltpu.async_copy(
                    mref.at[pl.ds(lbase + k0 * GCHUNK, GCHUNK)], rows0, sl0)
                l1 = pltpu.async_copy(
                    mref.at[pl.ds(lbase + k1 * GCHUNK, GCHUNK)], rows1, sl1)
                l0.wait()
                s0 = pltpu.async_copy(rows0, acc.at[idx_v.at[k0]], ss0,
                                      add=True)
                l1.wait()
                s1 = pltpu.async_copy(rows1, acc.at[idx_v.at[k1]], ss1,
                                      add=True)
                s0.wait()
                s1.wait()
                return carry

            lax.fori_loop(0, u // 2, pair, 0)
            if u % 2 == 1:
                kt = u - 1
                pltpu.sync_copy(
                    mref.at[pl.ds(lbase + kt * GCHUNK, GCHUNK)], rows0)
                pltpu.sync_copy(rows0, acc.at[idx_v.at[kt]], add=True)

        plsc.subcore_barrier()
        rbase = s * ROWS_PER_SUB
        pltpu.sync_copy(acc.at[pl.ds(rbase, ROWS_PER_SUB)],
                        out_hbm.at[c, pl.ds(rbase, ROWS_PER_SUB)])

    return body


def _sc_scatter(mess_list, dst3_list, units, init):
    zero_init = init is None
    max_u = max(units)
    scratch = [
        pltpu.VMEM((max_u, GCHUNK), jnp.int32),
        pltpu.VMEM((GCHUNK, D), _f32),
        pltpu.VMEM((GCHUNK, D), _f32),
        pltpu.VMEM_SHARED((NP, D), _f32),
        pltpu.SemaphoreType.DMA,
        pltpu.SemaphoreType.DMA,
        pltpu.SemaphoreType.DMA,
        pltpu.SemaphoreType.DMA,
    ]
    args = list(mess_list) + list(dst3_list)
    if zero_init:
        scratch.append(pltpu.VMEM((GCHUNK, D), _f32))
    else:
        args.append(init)
    return pl.kernel(
        _make_scatter_body(units, zero_init),
        out_type=jax.ShapeDtypeStruct((NC, NP, D), _f32),
        mesh=plsc.VectorSubcoreMesh(core_axis_name="c", subcore_axis_name="s"),
        scratch_types=scratch,
    )(*args)


def _edge_body(vi, vj, ef, rbf, wsrc, wdst, wee, wen, w2, bias, rbfw,
               new_e, mess):
    f32 = jnp.float32
    efb = ef[:]
    pvi = jnp.dot(vi[:], wsrc[:], preferred_element_type=f32)
    pvj = jnp.dot(vj[:], wdst[:], preferred_element_type=f32)
    basep = pvi + pvj
    pe = jnp.dot(efb, wee[:], preferred_element_type=f32)
    r = lax.dot_general(rbf[:], rbfw[:], (((0,), (0,)), ((), ())),
                        preferred_element_type=f32)

    e_h1 = jax.nn.silu(basep[:, 0:D] + pe[:, 0:D] + bias[0])
    e_g1 = jax.nn.silu(basep[:, D:2 * D] + pe[:, D:2 * D] + bias[2])
    e_h2 = jax.nn.silu(jnp.dot(e_h1, w2[0], preferred_element_type=f32)
                       + bias[1])
    e_g = jax.nn.sigmoid(jnp.dot(e_g1, w2[1], preferred_element_type=f32)
                         + bias[3])
    ne = efb + e_h2 * e_g * r[:, 0:D]
    new_e[:] = ne

    pne = jnp.dot(ne, wen[:], preferred_element_type=f32)
    n_h1 = jax.nn.silu(basep[:, 2 * D:3 * D] + pne[:, 0:D] + bias[4])
    n_g1 = jax.nn.silu(basep[:, 3 * D:4 * D] + pne[:, D:2 * D] + bias[6])
    n_h2 = jax.nn.silu(jnp.dot(n_h1, w2[2], preferred_element_type=f32)
                       + bias[5])
    n_g = jax.nn.sigmoid(jnp.dot(n_g1, w2[3], preferred_element_type=f32)
                         + bias[7])
    mess[:] = n_h2 * n_g * r[:, D:2 * D]


def _edge_body_alias(ne_in, vi, vj, ef, rbf, wsrc, wdst, wee, wen, w2,
                     bias, rbfw, new_e, mess):
    del ne_in
    _edge_body(vi, vj, ef, rbf, wsrc, wdst, wee, wen, w2, bias, rbfw,
               new_e, mess)


def _weight_specs():
    whole2 = lambda i: (0, 0)
    whole3 = lambda i: (0, 0, 0)
    return [
        pl.BlockSpec((D, 4 * D), whole2),
        pl.BlockSpec((D, 4 * D), whole2),
        pl.BlockSpec((D, 2 * D), whole2),
        pl.BlockSpec((D, 2 * D), whole2),
        pl.BlockSpec((4, D, D), whole3),
        pl.BlockSpec((8, D), whole2),
        pl.BlockSpec((9, 2 * D), whole2),
    ]


def _tc_edge_chunk(ne_buf, vi, vj, ef, rbft, weights, ch):
    u = UNITS[ch]
    ub = UBASE[ch]
    row = lambda i: (i, 0)
    rowc = lambda i, ub=ub: (ub + i, 0)
    colc = lambda i, ub=ub: (0, ub + i)
    first = ne_buf is None
    body = _edge_body if first else _edge_body_alias
    in_specs = [
        pl.BlockSpec((BLK, D), row),
        pl.BlockSpec((BLK, D), row),
        pl.BlockSpec((BLK, D), rowc),
        pl.BlockSpec((9, BLK), colc),
    ] + _weight_specs()
    args = [vi, vj, ef, rbft] + list(weights)
    aliases = {}
    if not first:
        in_specs = [pl.BlockSpec(memory_space=pltpu.MemorySpace.HBM)] \
            + in_specs
        args = [ne_buf] + args
        aliases = {0: 0}
    return pl.pallas_call(
        body,
        grid=(u,),
        in_specs=in_specs,
        out_specs=[
            pl.BlockSpec((BLK, D), rowc),
            pl.BlockSpec((BLK, D), row),
        ],
        out_shape=[
            jax.ShapeDtypeStruct((E, D), _f32),
            jax.ShapeDtypeStruct((u * UNIT, D), _f32),
        ],
        input_output_aliases=aliases,
        compiler_params=pltpu.CompilerParams(
            dimension_semantics=("arbitrary",)),
    )(*args)


def _combine_body(nf, p, out):
    out[:] = nf[:] + p[0] + p[1]


def _tc_combine(node_feat, partials):
    blk = 1000
    return pl.pallas_call(
        _combine_body,
        grid=(N // blk,),
        in_specs=[
            pl.BlockSpec((blk, D), lambda i: (i, 0)),
            pl.BlockSpec((NC, blk, D), lambda i: (0, i, 0)),
        ],
        out_specs=pl.BlockSpec((blk, D), lambda i: (i, 0)),
        out_shape=jax.ShapeDtypeStruct((N, D), _f32),
    )(node_feat, partials)


def kernel(node_feat, edge_feat, rbf, state_feat, edge_index,
           ew1, eb1, ew2, eb2, egw1, egb1, egw2, egb2, edge_rbf_w,
           nw1, nb1, nw2, nb2, ngw1, ngb1, ngw2, ngb2, node_rbf_w):
    src = edge_index[0].astype(jnp.int32)
    dst = edge_index[1].astype(jnp.int32)

    def chunk_idx(flat, ch):
        eb = UBASE[ch] * UNIT
        u = UNITS[ch]
        return lax.dynamic_slice(flat, (eb,), (u * UNIT,)).reshape(
            NW, u, GCHUNK)

    wsrc = jnp.concatenate(
        [ew1[:D], egw1[:D], nw1[:D], ngw1[:D]], axis=1)
    wdst = jnp.concatenate(
        [ew1[D:2 * D], egw1[D:2 * D], nw1[D:2 * D], ngw1[D:2 * D]], axis=1)
    wee = jnp.concatenate([ew1[2 * D:], egw1[2 * D:]], axis=1)
    wen = jnp.concatenate([nw1[2 * D:], ngw1[2 * D:]], axis=1)
    w2 = jnp.stack([ew2, egw2, nw2, ngw2])
    bias = jnp.stack([eb1, eb2, egb1, egb2, nb1, nb2, ngb1, ngb2])
    rbfw = jnp.concatenate([edge_rbf_w, node_rbf_w], axis=1)
    rbft = rbf.T
    weights = (wsrc, wdst, wee, wen, w2, bias, rbfw)

    ne_buf = None
    mess_chunks = []
    dst3s = []
    for ch in range(CH):
        vi, vj = _sc_gather(node_feat, chunk_idx(src, ch),
                            chunk_idx(dst, ch), UNITS[ch])
        ne_buf, m = _tc_edge_chunk(ne_buf, vi, vj,
                                   edge_feat, rbft, weights, ch)
        mess_chunks.append(m)
        dst3s.append(chunk_idx(dst, ch))

    pa = _sc_scatter(mess_chunks[:CH - 2], dst3s[:CH - 2],
                     UNITS[:CH - 2], None)
    pb = _sc_scatter(mess_chunks[CH - 2:CH - 1], dst3s[CH - 2:CH - 1],
                     UNITS[CH - 2:CH - 1], pa)
    pc = _sc_scatter(mess_chunks[CH - 1:], dst3s[CH - 1:],
                     UNITS[CH - 1:], pb)
    new_v = _tc_combine(node_feat, pc)
    return ne_buf, new_v, state_feat


# speedup vs baseline: 1.0686x; 1.0010x over previous
import jax
import jax.numpy as jnp
from jax import lax
from jax.experimental import pallas as pl
from jax.experimental.pallas import tpu as pltpu
from jax.experimental.pallas import tpu_sc as plsc

N = 10000
E = 320000
D = 128

NC = 2
NS = 16
NW = NC * NS

GCHUNK = 80
UNIT = NW * GCHUNK
UNITS = [9, 13, 19, 28, 37, 19]
UBASE = [sum(UNITS[:i]) for i in range(len(UNITS))]
CH = len(UNITS)

NP = 10240
ROWS_PER_SUB = NP // NS

BLK = UNIT

_f32 = jnp.float32


def _make_gather_body(u):
    def body(table, sidx3, didx3, vi_hbm, vj_hbm,
             idx_vs, idx_vd, rows0, rows1, sg0, sg1, sw0, sw1):
        c = lax.axis_index("c")
        s = lax.axis_index("s")
        wid = c * NS + s
        base = wid * u * GCHUNK
        pltpu.sync_copy(sidx3.at[wid], idx_vs)
        pltpu.sync_copy(didx3.at[wid], idx_vd)

        def half(idx_v, out_hbm):
            def pair(j, carry):
                k0 = 2 * j
                k1 = k0 + 1
                g0 = pltpu.async_copy(table.at[idx_v.at[k0]], rows0, sg0)
                g1 = pltpu.async_copy(table.at[idx_v.at[k1]], rows1, sg1)
                g0.wait()
                w0 = pltpu.async_copy(
                    rows0, out_hbm.at[pl.ds(base + k0 * GCHUNK, GCHUNK)],
                    sw0)
                g1.wait()
                w1 = pltpu.async_copy(
                    rows1, out_hbm.at[pl.ds(base + k1 * GCHUNK, GCHUNK)],
                    sw1)
                w0.wait()
                w1.wait()
                return carry

            lax.fori_loop(0, u // 2, pair, 0)
            if u % 2 == 1:
                kt = u - 1
                pltpu.async_copy(table.at[idx_v.at[kt]], rows0, sg0).wait()
                pltpu.sync_copy(
                    rows0, out_hbm.at[pl.ds(base + kt * GCHUNK, GCHUNK)])

        half(idx_vs, vi_hbm)
        half(idx_vd, vj_hbm)

    return body


def _sc_gather(node_feat, sidx3, didx3, u):
    ech = u * UNIT
    return pl.kernel(
        _make_gather_body(u),
        out_type=[jax.ShapeDtypeStruct((ech, D), _f32),
                  jax.ShapeDtypeStruct((ech, D), _f32)],
        mesh=plsc.VectorSubcoreMesh(core_axis_name="c", subcore_axis_name="s"),
        scratch_types=[
            pltpu.VMEM((u, GCHUNK), jnp.int32),
            pltpu.VMEM((u, GCHUNK), jnp.int32),
            pltpu.VMEM((GCHUNK, D), _f32),
            pltpu.VMEM((GCHUNK, D), _f32),
            pltpu.SemaphoreType.DMA,
            pltpu.SemaphoreType.DMA,
            pltpu.SemaphoreType.DMA,
            pltpu.SemaphoreType.DMA,
        ],
    )(node_feat, sidx3, didx3)


def _make_scatter_body(units, zero_init):
    nmess = len(units)

    def body(*refs):
        mess_refs = refs[:nmess]
        dst_refs = refs[nmess:2 * nmess]
        if zero_init:
            (out_hbm, idx_v, rows0, rows1, acc,
             sl0, sl1, ss0, ss1, zbuf) = refs[2 * nmess:]
        else:
            (init, out_hbm, idx_v, rows0, rows1, acc,
             sl0, sl1, ss0, ss1) = refs[2 * nmess:]
        c = lax.axis_index("c")
        s = lax.axis_index("s")

        if zero_init:
            zv = jnp.zeros((16,), _f32)

            def zrow(r, carry):
                for cc in range(D // 16):
                    zbuf[r, pl.ds(cc * 16, 16)] = zv
                return carry

            lax.fori_loop(0, GCHUNK, zrow, 0)

            def zcp(t, carry):
                pltpu.sync_copy(
                    zbuf,
                    acc.at[pl.ds(s * ROWS_PER_SUB + t * GCHUNK, GCHUNK)])
                return carry

            lax.fori_loop(0, ROWS_PER_SUB // GCHUNK, zcp, 0)
        else:
            @pl.when(s == 0)
            def _init():
                pltpu.sync_copy(init.at[c], acc)

        plsc.subcore_barrier()

        wid = c * NS + s
        for mi in range(nmess):
            u = units[mi]
            mref = mess_refs[mi]
            lbase = wid * u * GCHUNK
            pltpu.sync_copy(dst_refs[mi].at[wid], idx_v.at[pl.ds(0, u)])

            def pair(j, carry, mref=mref, lbase=lbase):
                k0 = 2 * j
                k1 = k0 + 1
                l0 = pltpu.async_copy(
                    mref.at[pl.ds(lbase + k0 * GCHUNK, GCHUNK)], rows0, sl0)
                l1 = pltpu.async_copy(
                    mref.at[pl.ds(lbase + k1 * GCHUNK, GCHUNK)], rows1, sl1)
                l0.wait()
                s0 = pltpu.async_copy(rows0, acc.at[idx_v.at[k0]], ss0,
                                      add=True)
                l1.wait()
                s1 = pltpu.async_copy(rows1, acc.at[idx_v.at[k1]], ss1,
                                      add=True)
                s0.wait()
                s1.wait()
                return carry

            lax.fori_loop(0, u // 2, pair, 0)
            if u % 2 == 1:
                kt = u - 1
                pltpu.sync_copy(
                    mref.at[pl.ds(lbase + kt * GCHUNK, GCHUNK)], rows0)
                pltpu.sync_copy(rows0, acc.at[idx_v.at[kt]], add=True)

        plsc.subcore_barrier()
        rbase = s * ROWS_PER_SUB
        pltpu.sync_copy(acc.at[pl.ds(rbase, ROWS_PER_SUB)],
                        out_hbm.at[c, pl.ds(rbase, ROWS_PER_SUB)])

    return body


def _sc_scatter(mess_list, dst3_list, units, init):
    zero_init = init is None
    max_u = max(units)
    scratch = [
        pltpu.VMEM((max_u, GCHUNK), jnp.int32),
        pltpu.VMEM((GCHUNK, D), _f32),
        pltpu.VMEM((GCHUNK, D), _f32),
        pltpu.VMEM_SHARED((NP, D), _f32),
        pltpu.SemaphoreType.DMA,
        pltpu.SemaphoreType.DMA,
        pltpu.SemaphoreType.DMA,
        pltpu.SemaphoreType.DMA,
    ]
    args = list(mess_list) + list(dst3_list)
    if zero_init:
        scratch.append(pltpu.VMEM((GCHUNK, D), _f32))
    else:
        args.append(init)
    return pl.kernel(
        _make_scatter_body(units, zero_init),
        out_type=jax.ShapeDtypeStruct((NC, NP, D), _f32),
        mesh=plsc.VectorSubcoreMesh(core_axis_name="c", subcore_axis_name="s"),
        scratch_types=scratch,
    )(*args)


def _edge_body(vi, vj, ef, rbf, wsrc, wdst, wee, wen, w2, bias, rbfw,
               new_e, mess):
    f32 = jnp.float32
    efb = ef[:]
    pvi = jnp.dot(vi[:], wsrc[:], preferred_element_type=f32)
    pvj = jnp.dot(vj[:], wdst[:], preferred_element_type=f32)
    basep = pvi + pvj
    pe = jnp.dot(efb, wee[:], preferred_element_type=f32)
    r = lax.dot_general(rbf[:], rbfw[:], (((0,), (0,)), ((), ())),
                        preferred_element_type=f32)

    e_h1 = jax.nn.silu(basep[:, 0:D] + pe[:, 0:D] + bias[0])
    e_g1 = jax.nn.silu(basep[:, D:2 * D] + pe[:, D:2 * D] + bias[2])
    e_h2 = jax.nn.silu(jnp.dot(e_h1, w2[0], preferred_element_type=f32)
                       + bias[1])
    e_g = jax.nn.sigmoid(jnp.dot(e_g1, w2[1], preferred_element_type=f32)
                         + bias[3])
    ne = efb + e_h2 * e_g * r[:, 0:D]
    new_e[:] = ne

    pne = jnp.dot(ne, wen[:], preferred_element_type=f32)
    n_h1 = jax.nn.silu(basep[:, 2 * D:3 * D] + pne[:, 0:D] + bias[4])
    n_g1 = jax.nn.silu(basep[:, 3 * D:4 * D] + pne[:, D:2 * D] + bias[6])
    n_h2 = jax.nn.silu(jnp.dot(n_h1, w2[2], preferred_element_type=f32)
                       + bias[5])
    n_g = jax.nn.sigmoid(jnp.dot(n_g1, w2[3], preferred_element_type=f32)
                         + bias[7])
    mess[:] = n_h2 * n_g * r[:, D:2 * D]


def _edge_body_alias(ne_in, vi, vj, ef, rbf, wsrc, wdst, wee, wen, w2,
                     bias, rbfw, new_e, mess):
    del ne_in
    _edge_body(vi, vj, ef, rbf, wsrc, wdst, wee, wen, w2, bias, rbfw,
               new_e, mess)


def _weight_specs():
    whole2 = lambda i: (0, 0)
    whole3 = lambda i: (0, 0, 0)
    return [
        pl.BlockSpec((D, 4 * D), whole2),
        pl.BlockSpec((D, 4 * D), whole2),
        pl.BlockSpec((D, 2 * D), whole2),
        pl.BlockSpec((D, 2 * D), whole2),
        pl.BlockSpec((4, D, D), whole3),
        pl.BlockSpec((8, D), whole2),
        pl.BlockSpec((9, 2 * D), whole2),
    ]


def _tc_edge_chunk(ne_buf, vi, vj, ef, rbft, weights, ch):
    u = UNITS[ch]
    ub = UBASE[ch]
    row = lambda i: (i, 0)
    rowc = lambda i, ub=ub: (ub + i, 0)
    colc = lambda i, ub=ub: (0, ub + i)
    first = ne_buf is None
    body = _edge_body if first else _edge_body_alias
    in_specs = [
        pl.BlockSpec((BLK, D), row),
        pl.BlockSpec((BLK, D), row),
        pl.BlockSpec((BLK, D), rowc),
        pl.BlockSpec((9, BLK), colc),
    ] + _weight_specs()
    args = [vi, vj, ef, rbft] + list(weights)
    aliases = {}
    if not first:
        in_specs = [pl.BlockSpec(memory_space=pltpu.MemorySpace.HBM)] \
            + in_specs
        args = [ne_buf] + args
        aliases = {0: 0}
    return pl.pallas_call(
        body,
        grid=(u,),
        in_specs=in_specs,
        out_specs=[
            pl.BlockSpec((BLK, D), rowc),
            pl.BlockSpec((BLK, D), row),
        ],
        out_shape=[
            jax.ShapeDtypeStruct((E, D), _f32),
            jax.ShapeDtypeStruct((u * UNIT, D), _f32),
        ],
        input_output_aliases=aliases,
        compiler_params=pltpu.CompilerParams(
            dimension_semantics=("parallel",)),
    )(*args)


def _combine_body(nf, p, out):
    out[:] = nf[:] + p[0] + p[1]


def _tc_combine(node_feat, partials):
    blk = 1000
    return pl.pallas_call(
        _combine_body,
        grid=(N // blk,),
        in_specs=[
            pl.BlockSpec((blk, D), lambda i: (i, 0)),
            pl.BlockSpec((NC, blk, D), lambda i: (0, i, 0)),
        ],
        out_specs=pl.BlockSpec((blk, D), lambda i: (i, 0)),
        out_shape=jax.ShapeDtypeStruct((N, D), _f32),
    )(node_feat, partials)


def kernel(node_feat, edge_feat, rbf, state_feat, edge_index,
           ew1, eb1, ew2, eb2, egw1, egb1, egw2, egb2, edge_rbf_w,
           nw1, nb1, nw2, nb2, ngw1, ngb1, ngw2, ngb2, node_rbf_w):
    src = edge_index[0].astype(jnp.int32)
    dst = edge_index[1].astype(jnp.int32)

    def chunk_idx(flat, ch):
        eb = UBASE[ch] * UNIT
        u = UNITS[ch]
        return lax.dynamic_slice(flat, (eb,), (u * UNIT,)).reshape(
            NW, u, GCHUNK)

    wsrc = jnp.concatenate(
        [ew1[:D], egw1[:D], nw1[:D], ngw1[:D]], axis=1)
    wdst = jnp.concatenate(
        [ew1[D:2 * D], egw1[D:2 * D], nw1[D:2 * D], ngw1[D:2 * D]], axis=1)
    wee = jnp.concatenate([ew1[2 * D:], egw1[2 * D:]], axis=1)
    wen = jnp.concatenate([nw1[2 * D:], ngw1[2 * D:]], axis=1)
    w2 = jnp.stack([ew2, egw2, nw2, ngw2])
    bias = jnp.stack([eb1, eb2, egb1, egb2, nb1, nb2, ngb1, ngb2])
    rbfw = jnp.concatenate([edge_rbf_w, node_rbf_w], axis=1)
    rbft = rbf.T
    weights = (wsrc, wdst, wee, wen, w2, bias, rbfw)

    ne_buf = None
    mess_chunks = []
    dst3s = []
    for ch in range(CH):
        vi, vj = _sc_gather(node_feat, chunk_idx(src, ch),
                            chunk_idx(dst, ch), UNITS[ch])
        ne_buf, m = _tc_edge_chunk(ne_buf, vi, vj,
                                   edge_feat, rbft, weights, ch)
        mess_chunks.append(m)
        dst3s.append(chunk_idx(dst, ch))

    pa = _sc_scatter(mess_chunks[:CH - 2], dst3s[:CH - 2],
                     UNITS[:CH - 2], None)
    pb = _sc_scatter(mess_chunks[CH - 2:CH - 1], dst3s[CH - 2:CH - 1],
                     UNITS[CH - 2:CH - 1], pa)
    pc = _sc_scatter(mess_chunks[CH - 1:], dst3s[CH - 1:],
                     UNITS[CH - 1:], pb)
    new_v = _tc_combine(node_feat, pc)
    return ne_buf, new_v, state_feat
